# baseline (speedup 1.0000x reference)
import functools

import numpy as np
import jax
import jax.numpy as jnp
from jax import lax
from jax.experimental import pallas as pl
from jax.experimental.pallas import tpu as pltpu
from jax.experimental.pallas import tpu_sc as plsc

_KS = 5
_SIGMA = 2.0
_HALF = _KS // 2

_NC = 2
_NS = 16
_NW = _NC * _NS
_L = 16
_IDX_CHUNK = 128


def _gauss1d():
    d = np.arange(-_HALF, _HALF + 1, dtype=np.float64)
    g = np.exp(-(d * d) / (2.0 * _SIGMA * _SIGMA))
    g = g / g.sum()
    return [float(v) for v in g]


_G = _gauss1d()


def _roll(v, shift, axis):
    if shift == 0:
        return v
    return jnp.roll(v, shift, axis)


def _yblur_body(in_ref, out_ref):
    h = in_ref.shape[1]
    n = h - 2 * _HALF
    acc = _G[0] * in_ref[:, pl.ds(0, n), :]
    for k in range(1, _KS):
        acc += _G[k] * in_ref[:, pl.ds(k, n), :]
    out_ref[:, pl.ds(_HALF, n), :] = acc


def _tr_body(in_ref, out_ref, t_ref):
    c, hb, w = in_ref.shape
    for hl in range(hb):
        t_ref[pl.ds(hl * w, w), :] = in_ref[:, hl, :].T
    rows = hb * w
    n = rows - 2 * _HALF
    acc = _G[0] * t_ref[pl.ds(0, n), :]
    for k in range(1, _KS):
        acc += _G[k] * t_ref[pl.ds(k, n), :]
    out_ref[pl.ds(_HALF, n), pl.ds(0, c)] = acc


def _make_gather(hw, c, n):
    bpw = -(-(-(-n // _NW)) // 8) * 8
    n_chunks = -(-bpw // _IDX_CHUNK)
    sizes = [_IDX_CHUNK] * (n_chunks - 1)
    sizes.append(bpw - _IDX_CHUNK * (n_chunks - 1))
    mesh = plsc.VectorSubcoreMesh(
        core_axis_name="c", subcore_axis_name="s",
        num_cores=_NC, num_subcores=_NS)

    @functools.partial(
        pl.kernel,
        mesh=mesh,
        compiler_params=pltpu.CompilerParams(use_tc_tiling_on_sc=False),
        out_type=jax.ShapeDtypeStruct((n, c), jnp.float32),
        scratch_types=[
            pltpu.VMEM((-(-bpw // _L) * _L,), jnp.int32),
            pltpu.VMEM((-(-bpw // _L) * _L,), jnp.int32),
            pltpu.VMEM((n_chunks, _IDX_CHUNK), jnp.int32),
            pltpu.VMEM((2, _IDX_CHUNK, c), jnp.float32),
            pltpu.SemaphoreType.DMA,
            pltpu.SemaphoreType.DMA,
        ],
    )
    def gather_k(table_hbm, x_hbm, y_hbm, out_hbm, xv, yv, idxv, rows,
                 sem0, sem1):
        wid = lax.axis_index("s") * _NC + lax.axis_index("c")
        base = jnp.minimum(wid * bpw, jnp.int32(n - bpw))
        sems = (sem0, sem1)
        pltpu.sync_copy(x_hbm.at[pl.ds(base, bpw)], xv.at[pl.ds(0, bpw)])
        pltpu.sync_copy(y_hbm.at[pl.ds(base, bpw)], yv.at[pl.ds(0, bpw)])
        lo = jnp.int32(_HALF)
        hi = jnp.int32(511 - _HALF)
        copies = [None] * n_chunks
        for j in range(n_chunks):
            for kk in range(-(-sizes[j] // _L)):
                lane0 = j * _IDX_CHUNK + kk * _L
                xi = jnp.clip(xv[pl.ds(lane0, _L)], lo, hi)
                yi = jnp.clip(yv[pl.ds(lane0, _L)], lo, hi)
                idxv[j, pl.ds(kk * _L, _L)] = yi * jnp.int32(512) + xi
            copies[j] = pltpu.async_copy(
                table_hbm.at[idxv.at[j]], rows.at[j % 2], sems[j % 2])
            if j >= 1:
                copies[j - 1].wait()
                pltpu.sync_copy(
                    rows.at[(j - 1) % 2].at[pl.ds(0, sizes[j - 1])],
                    out_hbm.at[pl.ds(base + (j - 1) * _IDX_CHUNK,
                                     sizes[j - 1])])
        j = n_chunks - 1
        copies[j].wait()
        pltpu.sync_copy(
            rows.at[j % 2].at[pl.ds(0, sizes[j])],
            out_hbm.at[pl.ds(base + j * _IDX_CHUNK, sizes[j])])

    return gather_k


def kernel(feature_map, keypoints):
    c, h, w = feature_map.shape
    n = keypoints.shape[0]

    cb = 4
    blurred = pl.pallas_call(
        _yblur_body,
        grid=(c // cb,),
        in_specs=[pl.BlockSpec((cb, h, w), lambda i: (i, 0, 0))],
        out_specs=pl.BlockSpec((cb, h, w), lambda i: (i, 0, 0)),
        out_shape=jax.ShapeDtypeStruct((c, h, w), jnp.float32),
    )(feature_map)

    hw = h * w
    hb = 8
    table = pl.pallas_call(
        _tr_body,
        grid=(h // hb,),
        in_specs=[pl.BlockSpec((c, hb, w), lambda i: (0, i, 0))],
        out_specs=pl.BlockSpec((hb * w, c), lambda i: (i, 0)),
        out_shape=jax.ShapeDtypeStruct((hw, c), jnp.float32),
        scratch_shapes=[pltpu.VMEM((hb * w, c), jnp.float32)],
    )(blurred)

    kp = keypoints.astype(jnp.int32)
    xs = kp[:, 0]
    ys = kp[:, 1]
    return _make_gather(hw, c, n)(table, xs, ys)


# speedup vs baseline: 1.5915x; 1.5915x over previous
import functools

import numpy as np
import jax
import jax.numpy as jnp
from jax import lax
from jax.experimental import pallas as pl
from jax.experimental.pallas import tpu as pltpu
from jax.experimental.pallas import tpu_sc as plsc

_KS = 5
_SIGMA = 2.0
_HALF = _KS // 2

_NC = 2
_NS = 16
_NW = _NC * _NS
_L = 16
_IDX_CHUNK = 128


def _gauss1d():
    d = np.arange(-_HALF, _HALF + 1, dtype=np.float64)
    g = np.exp(-(d * d) / (2.0 * _SIGMA * _SIGMA))
    g = g / g.sum()
    return [float(v) for v in g]


_G = _gauss1d()


def _roll(v, shift, axis):
    if shift == 0:
        return v
    return jnp.roll(v, shift, axis)


def _yblur_body(in_ref, out_ref):
    h = in_ref.shape[1]
    n = h - 2 * _HALF
    acc = _G[0] * in_ref[:, pl.ds(0, n), :]
    for k in range(1, _KS):
        acc += _G[k] * in_ref[:, pl.ds(k, n), :]
    out_ref[:, pl.ds(_HALF, n), :] = acc


def _tr_body(in_ref, out_ref, t_ref):
    c, hb, w = in_ref.shape
    for hl in range(hb):
        t_ref[pl.ds(hl * w, w), :] = in_ref[:, hl, :].T
    rows = hb * w
    n = rows - 2 * _HALF
    acc = _G[0] * t_ref[pl.ds(0, n), :]
    for k in range(1, _KS):
        acc += _G[k] * t_ref[pl.ds(k, n), :]
    out_ref[pl.ds(_HALF, n), pl.ds(0, c)] = acc


def _make_gather(hw, c, cp, n):
    bpw = -(-(-(-n // _NW)) // 8) * 8
    n_chunks = -(-bpw // _IDX_CHUNK)
    sizes = [_IDX_CHUNK] * (n_chunks - 1)
    sizes.append(bpw - _IDX_CHUNK * (n_chunks - 1))
    mesh = plsc.VectorSubcoreMesh(
        core_axis_name="c", subcore_axis_name="s",
        num_cores=_NC, num_subcores=_NS)

    @functools.partial(
        pl.kernel,
        mesh=mesh,
        compiler_params=pltpu.CompilerParams(use_tc_tiling_on_sc=True),
        out_type=jax.ShapeDtypeStruct((n, cp), jnp.float32),
        scratch_types=[
            pltpu.VMEM((-(-bpw // _L) * _L,), jnp.int32),
            pltpu.VMEM((-(-bpw // _L) * _L,), jnp.int32),
            pltpu.VMEM((n_chunks, _IDX_CHUNK), jnp.int32),
            pltpu.VMEM((2, _IDX_CHUNK, cp), jnp.float32),
            pltpu.SemaphoreType.DMA,
            pltpu.SemaphoreType.DMA,
        ],
    )
    def gather_k(table_hbm, x_hbm, y_hbm, out_hbm, xv, yv, idxv, rows,
                 sem0, sem1):
        wid = lax.axis_index("s") * _NC + lax.axis_index("c")
        base = jnp.minimum(wid * bpw, jnp.int32(n - bpw))
        sems = (sem0, sem1)
        pltpu.sync_copy(x_hbm.at[pl.ds(base, bpw)], xv.at[pl.ds(0, bpw)])
        pltpu.sync_copy(y_hbm.at[pl.ds(base, bpw)], yv.at[pl.ds(0, bpw)])
        lo = jnp.int32(_HALF)
        hi = jnp.int32(511 - _HALF)
        copies = [None] * n_chunks
        for j in range(n_chunks):
            for kk in range(-(-sizes[j] // _L)):
                lane0 = j * _IDX_CHUNK + kk * _L
                xi = jnp.clip(xv[pl.ds(lane0, _L)], lo, hi)
                yi = jnp.clip(yv[pl.ds(lane0, _L)], lo, hi)
                idxv[j, pl.ds(kk * _L, _L)] = yi * jnp.int32(512) + xi
            copies[j] = pltpu.async_copy(
                table_hbm.at[idxv.at[j]], rows.at[j % 2], sems[j % 2])
            if j >= 1:
                copies[j - 1].wait()
                pltpu.sync_copy(
                    rows.at[(j - 1) % 2].at[pl.ds(0, sizes[j - 1])],
                    out_hbm.at[pl.ds(base + (j - 1) * _IDX_CHUNK,
                                     sizes[j - 1])])
        j = n_chunks - 1
        copies[j].wait()
        pltpu.sync_copy(
            rows.at[j % 2].at[pl.ds(0, sizes[j])],
            out_hbm.at[pl.ds(base + j * _IDX_CHUNK, sizes[j])])

    return gather_k


def kernel(feature_map, keypoints):
    c, h, w = feature_map.shape
    n = keypoints.shape[0]

    cb = 4
    blurred = pl.pallas_call(
        _yblur_body,
        grid=(c // cb,),
        in_specs=[pl.BlockSpec((cb, h, w), lambda i: (i, 0, 0))],
        out_specs=pl.BlockSpec((cb, h, w), lambda i: (i, 0, 0)),
        out_shape=jax.ShapeDtypeStruct((c, h, w), jnp.float32),
    )(feature_map)

    hw = h * w
    cp = 256
    hb = 8
    table = pl.pallas_call(
        _tr_body,
        grid=(h // hb,),
        in_specs=[pl.BlockSpec((c, hb, w), lambda i: (0, i, 0))],
        out_specs=pl.BlockSpec((hb * w, cp), lambda i: (i, 0)),
        out_shape=jax.ShapeDtypeStruct((hw, cp), jnp.float32),
        scratch_shapes=[pltpu.VMEM((hb * w, c), jnp.float32)],
    )(blurred)

    kp = keypoints.astype(jnp.int32)
    xs = kp[:, 0]
    ys = kp[:, 1]
    return _make_gather(hw, c, cp, n)(table, xs, ys)[:, :c]
